# baseline (speedup 1.0000x reference)
import dataclasses
import functools

import jax
import jax.numpy as jnp
from jax import lax
from jax.experimental import pallas as pl
from jax.experimental.pallas import tpu as pltpu
from jax.experimental.pallas import tpu_sc as plsc

N = 10000
E = 320000
D = 128
NC = 2
NS = 16
NW = NC * NS
CH = 128
NCHUNK = 80
PH_CH = 32
F_SLOW = 0
F_FAST = 160
SLOW_CORE = 1
MAXCH = max(F_SLOW, F_FAST)
MAXPH = MAXCH // PH_CH
E_PAD = NW * NCHUNK * CH
ACC_ROWS = 10240
RPT = ACC_ROWS // NS
DUMMY = ACC_ROWS - 1

_mesh = plsc.VectorSubcoreMesh(core_axis_name="c", subcore_axis_name="s")

_sc_params = pltpu.CompilerParams()
if "needs_layout_passes" in pltpu.CompilerParams.__dataclass_fields__:
    _sc_params = dataclasses.replace(_sc_params, needs_layout_passes=False)


DEG_ROWS = 128
DRPT = DEG_ROWS // NS


@functools.partial(
    pl.kernel,
    out_type=jax.ShapeDtypeStruct((NC, DEG_ROWS, D), jnp.float32),
    mesh=_mesh,
    compiler_params=_sc_params,
    scratch_types=[
        pltpu.VMEM_SHARED((DEG_ROWS, D), jnp.float32),
        pltpu.VMEM((NCHUNK, CH), jnp.int32),
        pltpu.VMEM((DEG_ROWS, D), jnp.float32),
        pltpu.VMEM((1, DEG_ROWS), jnp.int32),
    ],
)
def _deg_sc(col_hbm, zeros_hbm, ar_hbm, degp_hbm, acc_sh, cidx, hist, ar):
    c = lax.axis_index("c")
    s = lax.axis_index("s")
    wid = c * NS + s
    base = s * DRPT
    pltpu.sync_copy(zeros_hbm.at[pl.ds(base, DRPT)],
                    acc_sh.at[pl.ds(base, DRPT)])
    pltpu.sync_copy(zeros_hbm, hist)
    pltpu.sync_copy(ar_hbm, ar)
    pltpu.sync_copy(col_hbm.at[wid], cidx)
    ones16 = jnp.ones((16,), jnp.float32)

    @pl.loop(0, NCHUNK)
    def _(j):
        @pl.loop(0, CH, step=16)
        def _(i):
            idx16 = cidx[j, pl.ds(i, 16)]
            hi = lax.shift_right_logical(idx16, 7)
            lo = lax.bitwise_and(idx16, 127)
            plsc.addupdate_scatter(hist, [hi, lo], ones16)

    plsc.subcore_barrier()
    pltpu.sync_copy(hist, acc_sh.at[ar.at[0]], add=True)
    plsc.subcore_barrier()
    pltpu.sync_copy(acc_sh.at[pl.ds(base, DRPT)],
                    degp_hbm.at[c].at[pl.ds(base, DRPT)])


@functools.partial(
    pl.kernel,
    out_type=jax.ShapeDtypeStruct((NC, ACC_ROWS, D), jnp.float32),
    mesh=_mesh,
    scratch_types=[
        pltpu.VMEM_SHARED((ACC_ROWS, D), jnp.float32),
        pltpu.VMEM((PH_CH, CH), jnp.int32),
        pltpu.VMEM((PH_CH, CH), jnp.int32),
        pltpu.VMEM((2, CH, D), jnp.float32),
        pltpu.SemaphoreType.DMA,
        pltpu.SemaphoreType.DMA,
    ],
)
def _scatter_sc(y_hbm, row_hbm, col_hbm, zeros_hbm, p_hbm,
                acc_sh, ridx, cidx, rows, sem0, sem1):
    c = lax.axis_index("c")
    s = lax.axis_index("s")
    wid = c * NS + s
    base = s * RPT
    pltpu.sync_copy(zeros_hbm.at[pl.ds(base, RPT)],
                    acc_sh.at[pl.ds(base, RPT)])
    plsc.subcore_barrier()

    nph = jnp.where(c == SLOW_CORE, F_SLOW // PH_CH, F_FAST // PH_CH)
    for ph in range(MAXPH):
        @pl.when(ph < nph)
        def _():
            pltpu.sync_copy(row_hbm.at[wid, pl.ds(ph * PH_CH, PH_CH)], ridx)
            pltpu.sync_copy(col_hbm.at[wid, pl.ds(ph * PH_CH, PH_CH)], cidx)
            pltpu.async_copy(y_hbm.at[ridx.at[0]], rows.at[0], sem0)

            @pl.loop(0, PH_CH, step=2)
            def _(j):
                pltpu.async_copy(y_hbm.at[ridx.at[j + 1]], rows.at[1], sem1)
                pltpu.make_async_copy(y_hbm.at[ridx.at[j]], rows.at[0],
                                      sem0).wait()
                pltpu.sync_copy(rows.at[0], acc_sh.at[cidx.at[j]], add=True)

                @pl.when(j + 2 < PH_CH)
                def _():
                    pltpu.async_copy(y_hbm.at[ridx.at[j + 2]], rows.at[0],
                                     sem0)

                pltpu.make_async_copy(y_hbm.at[ridx.at[j + 1]], rows.at[1],
                                      sem1).wait()
                pltpu.sync_copy(rows.at[1], acc_sh.at[cidx.at[j + 1]],
                                add=True)

    plsc.subcore_barrier()
    pltpu.sync_copy(acc_sh.at[pl.ds(base, RPT)],
                    p_hbm.at[c].at[pl.ds(base, RPT)])


def _mm_tc(x_ref, w_ref, o_ref):
    o_ref[...] = jnp.dot(x_ref[...], w_ref[...],
                         preferred_element_type=jnp.float32)


def _scale_tc(degp_ref, xw_ref, y_ref):
    deg = degp_ref[0, :N, :] + degp_ref[1, :N, :] + 1.0
    dis = lax.rsqrt(deg)
    y_ref[...] = xw_ref[...] * dis


def _mid_tc(degp_ref, p_ref, xw_ref, b1_ref, w2_ref, xw2_ref, y2_ref):
    deg = degp_ref[0, :N, :] + degp_ref[1, :N, :] + 1.0
    dis = lax.rsqrt(deg)
    dinv = dis * dis
    agg = p_ref[0, :N, :] + p_ref[1, :N, :]
    h = jnp.maximum(dis * agg + dinv * xw_ref[...] + b1_ref[...][None, :], 0.0)
    xw2 = jnp.dot(h, w2_ref[...], preferred_element_type=jnp.float32)
    xw2_ref[...] = xw2
    y2_ref[...] = xw2 * dis


def _out_tc(degp_ref, q_ref, xw2_ref, b2_ref, o_ref):
    deg = degp_ref[0, :N, :] + degp_ref[1, :N, :] + 1.0
    dis = lax.rsqrt(deg)
    dinv = dis * dis
    agg = q_ref[0, :N, :] + q_ref[1, :N, :]
    o_ref[...] = dis * agg + dinv * xw2_ref[...] + b2_ref[...][None, :]


_f32 = jnp.float32


_E_SLOW = NS * F_SLOW * CH


def _asym_layout(a, padval):
    slow = a[:_E_SLOW].reshape(NS, F_SLOW, CH)
    slow = jnp.concatenate(
        [slow, jnp.zeros((NS, MAXCH - F_SLOW, CH), jnp.int32)], axis=1)
    nfill = NS * F_FAST * CH - (E - _E_SLOW)
    fast = jnp.concatenate(
        [a[_E_SLOW:], jnp.full((nfill,), padval, jnp.int32)])
    fast = fast.reshape(NS, F_FAST, CH)
    blocks = (slow, fast) if SLOW_CORE == 0 else (fast, slow)
    return jnp.concatenate(blocks, axis=0)


def kernel(x, edge_index, W1, b1, W2, b2):
    ei = edge_index.astype(jnp.int32)
    row, col = ei[0], ei[1]
    npad = E_PAD - E
    colp_deg = jnp.concatenate(
        [col, jnp.full((npad,), DUMMY, jnp.int32)]).reshape(NW, NCHUNK, CH)
    rowp = _asym_layout(row, 0)
    colp = _asym_layout(col, DUMMY)
    zeros_deg = jnp.zeros((DEG_ROWS, D), _f32)
    ar_deg = jnp.arange(DEG_ROWS, dtype=jnp.int32).reshape(1, DEG_ROWS)
    zerosD = jnp.zeros((ACC_ROWS, D), _f32)

    degp = _deg_sc(colp_deg, zeros_deg, ar_deg).reshape(NC, DEG_ROWS * D, 1)
    xw1 = pl.pallas_call(
        _mm_tc, out_shape=jax.ShapeDtypeStruct((N, D), _f32))(x, W1)
    y1 = pl.pallas_call(
        _scale_tc, out_shape=jax.ShapeDtypeStruct((N, D), _f32))(degp, xw1)
    p = _scatter_sc(y1, rowp, colp, zerosD)
    xw2, y2 = pl.pallas_call(
        _mid_tc, out_shape=(jax.ShapeDtypeStruct((N, D), _f32),
                            jax.ShapeDtypeStruct((N, D), _f32)))(
        degp, p, xw1, b1, W2)
    q = _scatter_sc(y2, rowp, colp, zerosD)
    out = pl.pallas_call(
        _out_tc, out_shape=jax.ShapeDtypeStruct((N, D), _f32))(
        degp, q, xw2, b2)
    return out


# speedup vs baseline: 1.2726x; 1.2726x over previous
import dataclasses
import functools

import jax
import jax.numpy as jnp
from jax import lax
from jax.experimental import pallas as pl
from jax.experimental.pallas import tpu as pltpu
from jax.experimental.pallas import tpu_sc as plsc

N = 10000
E = 320000
D = 128
NC = 2
NS = 16
NW = NC * NS
CH = 128
NCHUNK = 80
PH_CH = 32
F_SLOW = 32
F_FAST = 128
SLOW_CORE = 0
MAXCH = max(F_SLOW, F_FAST)
MAXPH = MAXCH // PH_CH
E_PAD = NW * NCHUNK * CH
ACC_ROWS = 10240
RPT = ACC_ROWS // NS
DUMMY = ACC_ROWS - 1

_mesh = plsc.VectorSubcoreMesh(core_axis_name="c", subcore_axis_name="s")

_sc_params = pltpu.CompilerParams()
if "needs_layout_passes" in pltpu.CompilerParams.__dataclass_fields__:
    _sc_params = dataclasses.replace(_sc_params, needs_layout_passes=False)


DEG_ROWS = 128
DRPT = DEG_ROWS // NS


@functools.partial(
    pl.kernel,
    out_type=jax.ShapeDtypeStruct((NC, DEG_ROWS, D), jnp.float32),
    mesh=_mesh,
    compiler_params=_sc_params,
    scratch_types=[
        pltpu.VMEM_SHARED((DEG_ROWS, D), jnp.float32),
        pltpu.VMEM((NCHUNK, CH), jnp.int32),
        pltpu.VMEM((DEG_ROWS, D), jnp.float32),
        pltpu.VMEM((1, DEG_ROWS), jnp.int32),
    ],
)
def _deg_sc(col_hbm, zeros_hbm, ar_hbm, degp_hbm, acc_sh, cidx, hist, ar):
    c = lax.axis_index("c")
    s = lax.axis_index("s")
    wid = c * NS + s
    base = s * DRPT
    pltpu.sync_copy(zeros_hbm.at[pl.ds(base, DRPT)],
                    acc_sh.at[pl.ds(base, DRPT)])
    pltpu.sync_copy(zeros_hbm, hist)
    pltpu.sync_copy(ar_hbm, ar)
    pltpu.sync_copy(col_hbm.at[wid], cidx)
    ones16 = jnp.ones((16,), jnp.float32)

    @pl.loop(0, NCHUNK)
    def _(j):
        @pl.loop(0, CH, step=16)
        def _(i):
            idx16 = cidx[j, pl.ds(i, 16)]
            hi = lax.shift_right_logical(idx16, 7)
            lo = lax.bitwise_and(idx16, 127)
            plsc.addupdate_scatter(hist, [hi, lo], ones16)

    plsc.subcore_barrier()
    pltpu.sync_copy(hist, acc_sh.at[ar.at[0]], add=True)
    plsc.subcore_barrier()
    pltpu.sync_copy(acc_sh.at[pl.ds(base, DRPT)],
                    degp_hbm.at[c].at[pl.ds(base, DRPT)])


@functools.partial(
    pl.kernel,
    out_type=jax.ShapeDtypeStruct((NC, ACC_ROWS, D), jnp.float32),
    mesh=_mesh,
    scratch_types=[
        pltpu.VMEM_SHARED((ACC_ROWS, D), jnp.float32),
        pltpu.VMEM((PH_CH, CH), jnp.int32),
        pltpu.VMEM((PH_CH, CH), jnp.int32),
        pltpu.VMEM((2, CH, D), jnp.float32),
        pltpu.SemaphoreType.DMA,
        pltpu.SemaphoreType.DMA,
    ],
)
def _scatter_sc(y_hbm, row_hbm, col_hbm, zeros_hbm, p_hbm,
                acc_sh, ridx, cidx, rows, sem0, sem1):
    c = lax.axis_index("c")
    s = lax.axis_index("s")
    wid = c * NS + s
    base = s * RPT
    pltpu.sync_copy(zeros_hbm.at[pl.ds(base, RPT)],
                    acc_sh.at[pl.ds(base, RPT)])
    plsc.subcore_barrier()

    nph = jnp.where(c == SLOW_CORE, F_SLOW // PH_CH, F_FAST // PH_CH)
    for ph in range(MAXPH):
        @pl.when(ph < nph)
        def _():
            pltpu.sync_copy(row_hbm.at[wid, pl.ds(ph * PH_CH, PH_CH)], ridx)
            pltpu.sync_copy(col_hbm.at[wid, pl.ds(ph * PH_CH, PH_CH)], cidx)
            pltpu.async_copy(y_hbm.at[ridx.at[0]], rows.at[0], sem0)

            @pl.loop(0, PH_CH, step=2)
            def _(j):
                pltpu.async_copy(y_hbm.at[ridx.at[j + 1]], rows.at[1], sem1)
                pltpu.make_async_copy(y_hbm.at[ridx.at[j]], rows.at[0],
                                      sem0).wait()
                pltpu.sync_copy(rows.at[0], acc_sh.at[cidx.at[j]], add=True)

                @pl.when(j + 2 < PH_CH)
                def _():
                    pltpu.async_copy(y_hbm.at[ridx.at[j + 2]], rows.at[0],
                                     sem0)

                pltpu.make_async_copy(y_hbm.at[ridx.at[j + 1]], rows.at[1],
                                      sem1).wait()
                pltpu.sync_copy(rows.at[1], acc_sh.at[cidx.at[j + 1]],
                                add=True)

    plsc.subcore_barrier()
    pltpu.sync_copy(acc_sh.at[pl.ds(base, RPT)],
                    p_hbm.at[c].at[pl.ds(base, RPT)])


def _mm_tc(x_ref, w_ref, o_ref):
    o_ref[...] = jnp.dot(x_ref[...], w_ref[...],
                         preferred_element_type=jnp.float32)


def _scale_tc(degp_ref, xw_ref, y_ref):
    deg = degp_ref[0, :N, :] + degp_ref[1, :N, :] + 1.0
    dis = lax.rsqrt(deg)
    y_ref[...] = xw_ref[...] * dis


def _mid_tc(degp_ref, p_ref, xw_ref, b1_ref, w2_ref, xw2_ref, y2_ref):
    deg = degp_ref[0, :N, :] + degp_ref[1, :N, :] + 1.0
    dis = lax.rsqrt(deg)
    dinv = dis * dis
    agg = p_ref[0, :N, :] + p_ref[1, :N, :]
    h = jnp.maximum(dis * agg + dinv * xw_ref[...] + b1_ref[...][None, :], 0.0)
    xw2 = jnp.dot(h, w2_ref[...], preferred_element_type=jnp.float32)
    xw2_ref[...] = xw2
    y2_ref[...] = xw2 * dis


def _out_tc(degp_ref, q_ref, xw2_ref, b2_ref, o_ref):
    deg = degp_ref[0, :N, :] + degp_ref[1, :N, :] + 1.0
    dis = lax.rsqrt(deg)
    dinv = dis * dis
    agg = q_ref[0, :N, :] + q_ref[1, :N, :]
    o_ref[...] = dis * agg + dinv * xw2_ref[...] + b2_ref[...][None, :]


_f32 = jnp.float32


_E_SLOW = NS * F_SLOW * CH


def _asym_layout(a, padval):
    slow = a[:_E_SLOW].reshape(NS, F_SLOW, CH)
    slow = jnp.concatenate(
        [slow, jnp.zeros((NS, MAXCH - F_SLOW, CH), jnp.int32)], axis=1)
    nfill = NS * F_FAST * CH - (E - _E_SLOW)
    fast = jnp.concatenate(
        [a[_E_SLOW:], jnp.full((nfill,), padval, jnp.int32)])
    fast = fast.reshape(NS, F_FAST, CH)
    blocks = (slow, fast) if SLOW_CORE == 0 else (fast, slow)
    return jnp.concatenate(blocks, axis=0)


def kernel(x, edge_index, W1, b1, W2, b2):
    ei = edge_index.astype(jnp.int32)
    row, col = ei[0], ei[1]
    npad = E_PAD - E
    colp_deg = jnp.concatenate(
        [col, jnp.full((npad,), DUMMY, jnp.int32)]).reshape(NW, NCHUNK, CH)
    rowp = _asym_layout(row, 0)
    colp = _asym_layout(col, DUMMY)
    zeros_deg = jnp.zeros((DEG_ROWS, D), _f32)
    ar_deg = jnp.arange(DEG_ROWS, dtype=jnp.int32).reshape(1, DEG_ROWS)
    zerosD = jnp.zeros((ACC_ROWS, D), _f32)

    degp = _deg_sc(colp_deg, zeros_deg, ar_deg).reshape(NC, DEG_ROWS * D, 1)
    xw1 = pl.pallas_call(
        _mm_tc, out_shape=jax.ShapeDtypeStruct((N, D), _f32))(x, W1)
    y1 = pl.pallas_call(
        _scale_tc, out_shape=jax.ShapeDtypeStruct((N, D), _f32))(degp, xw1)
    p = _scatter_sc(y1, rowp, colp, zerosD)
    xw2, y2 = pl.pallas_call(
        _mid_tc, out_shape=(jax.ShapeDtypeStruct((N, D), _f32),
                            jax.ShapeDtypeStruct((N, D), _f32)))(
        degp, p, xw1, b1, W2)
    q = _scatter_sc(y2, rowp, colp, zerosD)
    out = pl.pallas_call(
        _out_tc, out_shape=jax.ShapeDtypeStruct((N, D), _f32))(
        degp, q, xw2, b2)
    return out
